# baseline (speedup 1.0000x reference)
import jax
import jax.numpy as jnp
from jax import lax
from jax.experimental import pallas as pl
from jax.experimental.pallas import tpu as pltpu
from jax.experimental.pallas import tpu_sc as plsc

B = 4; C = 3; H = 16; W = 16; D = 16
K = 16
DEPTH = 4
S = 2 ** DEPTH
HO = WO = DO = 14
P = HO * WO * DO
P_PAD = 2752
HALF = P_PAD // 2
CHUNKS = HALF // 16
XB = C * H * W * D
XWORDS = B * XB
WOFF = (0, 4096, 6144, 7168, 7680)
WWORDS = 7936
NL = 16

NOFF = (0, 16, 24, 28, 30)


def _sc_body(x_hbm, pa_hbm, pb_hbm, w_hbm, out_hbm,
             x_v, pa_v, pb_v, w_v, coef_v, out_v):
    k = lax.axis_index("s")
    half = lax.axis_index("c")
    pltpu.sync_copy(x_hbm, x_v)
    pltpu.sync_copy(pa_hbm, pa_v)
    pltpu.sync_copy(pb_hbm, pb_v)
    pltpu.sync_copy(w_hbm, w_v)

    i32 = jnp.int32

    def leaf_offsets(p_v):
        offs = []
        for s in range(S):
            base = k * (S * 4) + s * 4
            ph = plsc.load_gather(p_v, [jnp.full((NL,), base + 0, i32)])
            pw = plsc.load_gather(p_v, [jnp.full((NL,), base + 1, i32)])
            pd = plsc.load_gather(p_v, [jnp.full((NL,), base + 2, i32)])
            pc = plsc.load_gather(p_v, [jnp.full((NL,), base + 3, i32)])
            offs.append(pc * 4096 + ph * 256 + pw * 16 + pd)
        return offs

    offs_a = [jnp.max(v) for v in leaf_offsets(pa_v)]
    offs_b = [jnp.max(v) for v in leaf_offsets(pb_v)]

    f32 = jnp.float32
    gi = lax.iota(i32, NL)
    t11 = gi & 1
    t10 = (gi >> 1) & 1
    t01 = (gi >> 2) & 1
    t00 = (gi >> 3) & 1
    m0 = t00.astype(f32)
    m1 = (t10 - t00).astype(f32)
    m2 = (t01 - t00).astype(f32)
    m3 = (t11 - t10 - t01 + t00).astype(f32)
    for lvl in range(DEPTH + 1):
        for t in range(16 >> lvl):
            wv = w_v[pl.ds(WOFF[lvl] + (t * K + k) * 16, 16)]
            e = jnp.exp(wv - jnp.max(wv))
            pvec = e / jnp.sum(e)
            nid = NOFF[lvl] + t
            for j, m in enumerate((m0, m1, m2, m3)):
                coef_v[pl.ds((nid * 4 + j) * 16, 16)] = jnp.full(
                    (NL,), jnp.sum(pvec * m), f32)

    iota = lax.iota(i32, NL)

    def load_cf(lvl, t):
        nid = NOFF[lvl] + t
        return [coef_v[pl.ds((nid * 4 + j) * 16, 16)] for j in range(4)]

    def node(a2, b2, cf):
        c0, c1, c2, c3 = cf
        return (c0 + c1 * a2) + b2 * (c2 + c3 * a2)

    @plsc.parallel_loop(0, CHUNKS, unroll=4)
    def chunk(ci):
        p0 = half * HALF + ci * NL
        pvec = jnp.minimum(p0 + iota, P - 1)
        t = pvec // 14
        od = pvec - t * 14
        oh = t // 14
        ow = t - oh * 14
        lin = oh * 256 + ow * 16 + od
        stack = []
        cnt = [0] * (DEPTH + 1)
        for s in range(S):
            cf = load_cf(0, s)
            ia = lin + jnp.full((NL,), offs_a[s], i32)
            ib = lin + jnp.full((NL,), offs_b[s], i32)
            vals = []
            for b in range(B):
                xb = x_v.at[pl.ds(b * XB, XB)]
                vals.append(node(plsc.load_gather(xb, [ia]),
                                 plsc.load_gather(xb, [ib]), cf))
            entry = (0, vals)
            while stack and stack[-1][0] == entry[0]:
                lvl = entry[0] + 1
                cf = load_cf(lvl, cnt[lvl])
                cnt[lvl] += 1
                left = stack.pop()[1]
                entry = (lvl, [node(left[b], entry[1][b], cf)
                               for b in range(B)])
            stack.append(entry)
        vals = stack[0][1]
        for b in range(B):
            out_v[pl.ds(b * HALF + ci * NL, NL)] = vals[b]

    pltpu.sync_copy(out_v, out_hbm.at[k, half])


def kernel(x, pairs_a, pairs_b, W0, W1, W2, W3, W4):
    xf = x.reshape(-1)
    paf = pairs_a.reshape(-1)
    pbf = pairs_b.reshape(-1)
    wf = jnp.concatenate([W0.reshape(-1), W1.reshape(-1), W2.reshape(-1),
                          W3.reshape(-1), W4.reshape(-1)])
    mesh = plsc.VectorSubcoreMesh(core_axis_name="c", subcore_axis_name="s",
                                  num_cores=2, num_subcores=16)
    out = pl.kernel(
        _sc_body,
        out_type=jax.ShapeDtypeStruct((K, 2, B * HALF), jnp.float32),
        mesh=mesh,
        compiler_params=pltpu.CompilerParams(needs_layout_passes=False),
        scratch_types=[
            pltpu.VMEM((XWORDS,), jnp.float32),
            pltpu.VMEM((K * S * 4,), jnp.int32),
            pltpu.VMEM((K * S * 4,), jnp.int32),
            pltpu.VMEM((WWORDS,), jnp.float32),
            pltpu.VMEM((31 * 4 * NL,), jnp.float32),
            pltpu.VMEM((B * HALF,), jnp.float32),
        ],
    )(xf, paf, pbf, wf)
    out = out.reshape(K, 2, B, HALF).transpose(2, 0, 1, 3).reshape(B, K, P_PAD)
    return out[:, :, :P].reshape(B, K, HO, WO, DO)


# speedup vs baseline: 1.0125x; 1.0125x over previous
import jax
import jax.numpy as jnp
from jax import lax
from jax.experimental import pallas as pl
from jax.experimental.pallas import tpu as pltpu
from jax.experimental.pallas import tpu_sc as plsc

B = 4; C = 3; H = 16; W = 16; D = 16
K = 16
DEPTH = 4
S = 2 ** DEPTH
HO = WO = DO = 14
P = HO * WO * DO
P_PAD = 2752
HALF = P_PAD // 2
CHUNKS = HALF // 16
XB = C * H * W * D
XWORDS = B * XB
WOFF = (0, 4096, 6144, 7168, 7680)
WWORDS = 7936
NL = 16

NOFF = (0, 16, 24, 28, 30)


def _sc_body(x_hbm, pa_hbm, pb_hbm, w_hbm, out_hbm,
             x_v, pa_v, pb_v, w_v, coef_v, out_v):
    k = lax.axis_index("s")
    half = lax.axis_index("c")
    pltpu.sync_copy(x_hbm, x_v)
    pltpu.sync_copy(pa_hbm, pa_v)
    pltpu.sync_copy(pb_hbm, pb_v)
    pltpu.sync_copy(w_hbm, w_v)

    i32 = jnp.int32

    def leaf_offsets(p_v):
        offs = []
        for s in range(S):
            base = k * (S * 4) + s * 4
            ph = plsc.load_gather(p_v, [jnp.full((NL,), base + 0, i32)])
            pw = plsc.load_gather(p_v, [jnp.full((NL,), base + 1, i32)])
            pd = plsc.load_gather(p_v, [jnp.full((NL,), base + 2, i32)])
            pc = plsc.load_gather(p_v, [jnp.full((NL,), base + 3, i32)])
            offs.append(pc * 4096 + ph * 256 + pw * 16 + pd)
        return offs

    offs_a = [jnp.max(v) for v in leaf_offsets(pa_v)]
    offs_b = [jnp.max(v) for v in leaf_offsets(pb_v)]

    f32 = jnp.float32
    gi = lax.iota(i32, NL)
    t11 = gi & 1
    t10 = (gi >> 1) & 1
    t01 = (gi >> 2) & 1
    t00 = (gi >> 3) & 1
    m0 = t00.astype(f32)
    m1 = (t10 - t00).astype(f32)
    m2 = (t01 - t00).astype(f32)
    m3 = (t11 - t10 - t01 + t00).astype(f32)
    for lvl in range(DEPTH + 1):
        for t in range(16 >> lvl):
            wv = w_v[pl.ds(WOFF[lvl] + (t * K + k) * 16, 16)]
            e = jnp.exp(wv - jnp.max(wv))
            pvec = e / jnp.sum(e)
            nid = NOFF[lvl] + t
            for j, m in enumerate((m0, m1, m2, m3)):
                coef_v[pl.ds((nid * 4 + j) * 16, 16)] = jnp.full(
                    (NL,), jnp.sum(pvec * m), f32)

    iota = lax.iota(i32, NL)

    def load_cf(lvl, t):
        nid = NOFF[lvl] + t
        return [coef_v[pl.ds((nid * 4 + j) * 16, 16)] for j in range(4)]

    def node(a2, b2, cf):
        c0, c1, c2, c3 = cf
        return (c0 + c1 * a2) + b2 * (c2 + c3 * a2)

    @plsc.parallel_loop(0, CHUNKS // 2, unroll=2)
    def chunk(q):
        lins = []
        for h in range(2):
            p0 = half * HALF + (2 * q + h) * NL
            pvec = jnp.minimum(p0 + iota, P - 1)
            t = pvec // 14
            od = pvec - t * 14
            oh = t // 14
            ow = t - oh * 14
            lins.append(oh * 256 + ow * 16 + od)
        stack = []
        cnt = [0] * (DEPTH + 1)
        for s in range(S):
            cf = load_cf(0, s)
            bca = jnp.full((NL,), offs_a[s], i32)
            bcb = jnp.full((NL,), offs_b[s], i32)
            vals = []
            for h in range(2):
                ia = lins[h] + bca
                ib = lins[h] + bcb
                for b in range(B):
                    xb = x_v.at[pl.ds(b * XB, XB)]
                    vals.append(node(plsc.load_gather(xb, [ia]),
                                     plsc.load_gather(xb, [ib]), cf))
            entry = (0, vals)
            while stack and stack[-1][0] == entry[0]:
                lvl = entry[0] + 1
                cf = load_cf(lvl, cnt[lvl])
                cnt[lvl] += 1
                left = stack.pop()[1]
                entry = (lvl, [node(left[j], entry[1][j], cf)
                               for j in range(2 * B)])
            stack.append(entry)
        vals = stack[0][1]
        for h in range(2):
            for b in range(B):
                out_v[pl.ds(b * HALF + (2 * q + h) * NL, NL)] = vals[h * B + b]

    pltpu.sync_copy(out_v, out_hbm.at[k, half])


def kernel(x, pairs_a, pairs_b, W0, W1, W2, W3, W4):
    xf = x.reshape(-1)
    paf = pairs_a.reshape(-1)
    pbf = pairs_b.reshape(-1)
    wf = jnp.concatenate([W0.reshape(-1), W1.reshape(-1), W2.reshape(-1),
                          W3.reshape(-1), W4.reshape(-1)])
    mesh = plsc.VectorSubcoreMesh(core_axis_name="c", subcore_axis_name="s",
                                  num_cores=2, num_subcores=16)
    out = pl.kernel(
        _sc_body,
        out_type=jax.ShapeDtypeStruct((K, 2, B * HALF), jnp.float32),
        mesh=mesh,
        compiler_params=pltpu.CompilerParams(needs_layout_passes=False),
        scratch_types=[
            pltpu.VMEM((XWORDS,), jnp.float32),
            pltpu.VMEM((K * S * 4,), jnp.int32),
            pltpu.VMEM((K * S * 4,), jnp.int32),
            pltpu.VMEM((WWORDS,), jnp.float32),
            pltpu.VMEM((31 * 4 * NL,), jnp.float32),
            pltpu.VMEM((B * HALF,), jnp.float32),
        ],
    )(xf, paf, pbf, wf)
    out = out.reshape(K, 2, B, HALF).transpose(2, 0, 1, 3).reshape(B, K, P_PAD)
    return out[:, :, :P].reshape(B, K, HO, WO, DO)


# speedup vs baseline: 1.0628x; 1.0497x over previous
import jax
import jax.numpy as jnp
from jax import lax
from jax.experimental import pallas as pl
from jax.experimental.pallas import tpu as pltpu
from jax.experimental.pallas import tpu_sc as plsc

B = 4; C = 3; H = 16; W = 16; D = 16
K = 16
DEPTH = 4
S = 2 ** DEPTH
HO = WO = DO = 14
P = HO * WO * DO
P_PAD = 2752
HALF = P_PAD // 2
CHUNKS = HALF // 16
XB = C * H * W * D
XWORDS = B * XB
WOFF = (0, 4096, 6144, 7168, 7680)
WWORDS = 7936
NL = 16

NOFF = (0, 16, 24, 28, 30)


def _sc_body(x_hbm, pa_hbm, pb_hbm, w_hbm, out_hbm,
             x_v, pa_v, pb_v, w_v, coef_v, out_v):
    k = lax.axis_index("s")
    half = lax.axis_index("c")
    pltpu.sync_copy(x_hbm, x_v)
    pltpu.sync_copy(pa_hbm, pa_v)
    pltpu.sync_copy(pb_hbm, pb_v)
    pltpu.sync_copy(w_hbm, w_v)

    i32 = jnp.int32

    def leaf_offsets(p_v):
        offs = []
        for s in range(S):
            base = k * (S * 4) + s * 4
            ph = plsc.load_gather(p_v, [jnp.full((NL,), base + 0, i32)])
            pw = plsc.load_gather(p_v, [jnp.full((NL,), base + 1, i32)])
            pd = plsc.load_gather(p_v, [jnp.full((NL,), base + 2, i32)])
            pc = plsc.load_gather(p_v, [jnp.full((NL,), base + 3, i32)])
            offs.append(pc * 4096 + ph * 256 + pw * 16 + pd)
        return offs

    offs_a = [jnp.max(v) for v in leaf_offsets(pa_v)]
    offs_b = [jnp.max(v) for v in leaf_offsets(pb_v)]

    f32 = jnp.float32
    gi = lax.iota(i32, NL)
    t11 = gi & 1
    t10 = (gi >> 1) & 1
    t01 = (gi >> 2) & 1
    t00 = (gi >> 3) & 1
    m0 = t00.astype(f32)
    m1 = (t10 - t00).astype(f32)
    m2 = (t01 - t00).astype(f32)
    m3 = (t11 - t10 - t01 + t00).astype(f32)
    bf16 = jnp.bfloat16
    for lvl in range(DEPTH + 1):
        for t in range(16 >> lvl):
            wv = w_v[pl.ds(WOFF[lvl] + (t * K + k) * 16, 16)]
            e = jnp.exp(wv - jnp.max(wv))
            pvec = e / jnp.sum(e)
            nid = NOFF[lvl] + t
            for j, m in enumerate((m0, m1, m2, m3)):
                c16 = jnp.full((NL,), jnp.sum(pvec * m), f32)
                cpk = plsc.pack(c16, c16, format=plsc.PackFormat.INTERLEAVED)
                coef_v[pl.ds((nid * 4 + j) * NL, NL)] = plsc.bitcast(cpk, i32)

    iota = lax.iota(i32, NL)

    def load_cf(lvl, t):
        nid = NOFF[lvl] + t
        return [plsc.bitcast(coef_v[pl.ds((nid * 4 + j) * NL, NL)], bf16)
                for j in range(4)]

    def node(a2, b2, cf):
        c0, c1, c2, c3 = cf
        return (c0 + c1 * a2) + b2 * (c2 + c3 * a2)

    @plsc.parallel_loop(0, CHUNKS // 2, unroll=2)
    def chunk(q):
        lins = []
        for h in range(2):
            p0 = half * HALF + (2 * q + h) * NL
            pvec = jnp.minimum(p0 + iota, P - 1)
            t = pvec // 14
            od = pvec - t * 14
            oh = t // 14
            ow = t - oh * 14
            lins.append(oh * 256 + ow * 16 + od)
        stack = []
        cnt = [0] * (DEPTH + 1)
        fmt = plsc.PackFormat.INTERLEAVED
        for s in range(S):
            cf = load_cf(0, s)
            bca = jnp.full((NL,), offs_a[s], i32)
            bcb = jnp.full((NL,), offs_b[s], i32)
            ia = [lins[h] + bca for h in range(2)]
            ib = [lins[h] + bcb for h in range(2)]
            vals = []
            for b in range(B):
                xb = x_v.at[pl.ds(b * XB, XB)]
                av = plsc.pack(plsc.load_gather(xb, [ia[0]]),
                               plsc.load_gather(xb, [ia[1]]), format=fmt)
                bv = plsc.pack(plsc.load_gather(xb, [ib[0]]),
                               plsc.load_gather(xb, [ib[1]]), format=fmt)
                vals.append(node(av, bv, cf))
            entry = (0, vals)
            while stack and stack[-1][0] == entry[0]:
                lvl = entry[0] + 1
                cf = load_cf(lvl, cnt[lvl])
                cnt[lvl] += 1
                left = stack.pop()[1]
                entry = (lvl, [node(left[j], entry[1][j], cf)
                               for j in range(B)])
            stack.append(entry)
        vals = stack[0][1]
        for b in range(B):
            v0, v1 = plsc.unpack(vals[b], format=fmt)
            out_v[pl.ds(b * HALF + (2 * q + 0) * NL, NL)] = v0
            out_v[pl.ds(b * HALF + (2 * q + 1) * NL, NL)] = v1

    pltpu.sync_copy(out_v, out_hbm.at[k, half])


def kernel(x, pairs_a, pairs_b, W0, W1, W2, W3, W4):
    xf = x.reshape(-1)
    paf = pairs_a.reshape(-1)
    pbf = pairs_b.reshape(-1)
    wf = jnp.concatenate([W0.reshape(-1), W1.reshape(-1), W2.reshape(-1),
                          W3.reshape(-1), W4.reshape(-1)])
    mesh = plsc.VectorSubcoreMesh(core_axis_name="c", subcore_axis_name="s",
                                  num_cores=2, num_subcores=16)
    out = pl.kernel(
        _sc_body,
        out_type=jax.ShapeDtypeStruct((K, 2, B * HALF), jnp.float32),
        mesh=mesh,
        compiler_params=pltpu.CompilerParams(needs_layout_passes=False),
        scratch_types=[
            pltpu.VMEM((XWORDS,), jnp.float32),
            pltpu.VMEM((K * S * 4,), jnp.int32),
            pltpu.VMEM((K * S * 4,), jnp.int32),
            pltpu.VMEM((WWORDS,), jnp.float32),
            pltpu.VMEM((31 * 4 * NL,), jnp.int32),
            pltpu.VMEM((B * HALF,), jnp.float32),
        ],
    )(xf, paf, pbf, wf)
    out = out.reshape(K, 2, B, HALF).transpose(2, 0, 1, 3).reshape(B, K, P_PAD)
    return out[:, :, :P].reshape(B, K, HO, WO, DO)
